# Initial kernel scaffold; baseline (speedup 1.0000x reference)
import functools

import jax
import jax.numpy as jnp
from jax import lax
from jax.experimental import pallas as pl
from jax.experimental.pallas import tpu as pltpu
from jax.experimental.pallas import tpu_sc as plsc

N = 10000
E = 320000
E_PAD = 327680
NW = 32
EPT = E_PAD // NW
NOUTER = EPT // 1024
RPT = N // 16
RCH = 125
RB = 2000


def _mesh():
    return plsc.VectorSubcoreMesh(core_axis_name="c", subcore_axis_name="s")


@functools.partial(
    pl.kernel,
    out_type=jax.ShapeDtypeStruct((2, N, 16), jnp.float32),
    mesh=_mesh(),
    scratch_types=[
        pltpu.VMEM((8, 128), jnp.int32),
        pltpu.VMEM((128, 16), jnp.float32),
        pltpu.VMEM((RCH, 16), jnp.float32),
        pltpu.VMEM_SHARED((N + 8, 16), jnp.float32),
    ],
)
def _deg(dst_hbm, out_hbm, didx, ones, stage, shared):
    c = lax.axis_index("c")
    s = lax.axis_index("s")
    t = c * 16 + s
    r0 = s * RPT

    def fill_ones(i, _):
        ones[i, :] = jnp.full((16,), 1.0, jnp.float32)
        return 0

    lax.fori_loop(0, 128, fill_ones, 0)

    def fill_zero(i, _):
        stage[i, :] = jnp.zeros((16,), jnp.float32)
        return 0

    lax.fori_loop(0, RCH, fill_zero, 0)

    def zero_blk(k, _):
        pltpu.sync_copy(stage, shared.at[pl.ds(r0 + k * RCH, RCH)])
        return 0

    lax.fori_loop(0, RPT // RCH, zero_blk, 0)
    plsc.subcore_barrier()

    def blk(b, _):
        row0 = t * (EPT // 128) + b * 8
        pltpu.sync_copy(dst_hbm.at[pl.ds(row0, 8)], didx)
        for j in range(8):
            pltpu.sync_copy(ones, shared.at[didx.at[j]], add=True)
        return 0

    lax.fori_loop(0, NOUTER, blk, 0)
    plsc.subcore_barrier()

    def rd(k, _):
        pltpu.sync_copy(shared.at[pl.ds(r0 + k * RCH, RCH)], stage)
        pltpu.sync_copy(stage, out_hbm.at[c, pl.ds(r0 + k * RCH, RCH)])
        return 0

    lax.fori_loop(0, RPT // RCH, rd, 0)


def _make_agg(D):
    @functools.partial(
        pl.kernel,
        out_type=jax.ShapeDtypeStruct((2, N, D), jnp.float32),
        mesh=_mesh(),
        scratch_types=[
            pltpu.VMEM((8, 128), jnp.int32),
            pltpu.VMEM((8, 128), jnp.int32),
            pltpu.VMEM((128, D), jnp.float32),
            pltpu.VMEM((RCH, D), jnp.float32),
            pltpu.VMEM_SHARED((N + 8, D), jnp.float32),
            pltpu.SemaphoreType.DMA,
        ],
    )
    def agg(g_hbm, src_hbm, dst_hbm, out_hbm, sidx, didx, rows, stage, shared, sem):
        c = lax.axis_index("c")
        s = lax.axis_index("s")
        t = c * 16 + s
        r0 = s * RPT

        def init_blk(k, _):
            pltpu.sync_copy(g_hbm.at[pl.ds(r0 + k * RCH, RCH)], stage)
            pltpu.sync_copy(stage, shared.at[pl.ds(r0 + k * RCH, RCH)])
            return 0

        lax.fori_loop(0, RPT // RCH, init_blk, 0)
        plsc.subcore_barrier()

        def blk(b, _):
            row0 = t * (EPT // 128) + b * 8
            pltpu.sync_copy(src_hbm.at[pl.ds(row0, 8)], sidx)
            pltpu.sync_copy(dst_hbm.at[pl.ds(row0, 8)], didx)
            for j in range(8):
                pltpu.async_copy(g_hbm.at[sidx.at[j]], rows, sem).wait()
                pltpu.sync_copy(rows, shared.at[didx.at[j]], add=True)
            return 0

        lax.fori_loop(0, NOUTER, blk, 0)
        plsc.subcore_barrier()

        def rd(k, _):
            pltpu.sync_copy(shared.at[pl.ds(r0 + k * RCH, RCH)], stage)
            pltpu.sync_copy(stage, out_hbm.at[c, pl.ds(r0 + k * RCH, RCH)])
            return 0

        lax.fori_loop(0, RPT // RCH, rd, 0)

    return agg


_agg128 = _make_agg(128)
_agg16 = _make_agg(16)


def _first_body(deg_ref, x_ref, y_ref, w0_ref, lw0_ref, dinv_ref, gh_ref, gl_ref):
    d = deg_ref[0, :, 0:1] + deg_ref[1, :, 0:1] + 1.0
    dinv = lax.rsqrt(d)
    dinv_ref[...] = dinv
    gh_ref[...] = jnp.dot(x_ref[...], w0_ref[...],
                          preferred_element_type=jnp.float32) * dinv
    gl_ref[...] = jnp.dot(y_ref[...], lw0_ref[...],
                          preferred_element_type=jnp.float32) * dinv


def _tc_first(deg2, x, y, w0, lw0):
    grid = (N // RB,)
    return pl.pallas_call(
        _first_body,
        grid=grid,
        in_specs=[
            pl.BlockSpec((2, RB, 16), lambda i: (0, i, 0)),
            pl.BlockSpec((RB, 128), lambda i: (i, 0)),
            pl.BlockSpec((RB, 16), lambda i: (i, 0)),
            pl.BlockSpec((128, 128), lambda i: (0, 0)),
            pl.BlockSpec((16, 16), lambda i: (0, 0)),
        ],
        out_specs=[
            pl.BlockSpec((RB, 1), lambda i: (i, 0)),
            pl.BlockSpec((RB, 128), lambda i: (i, 0)),
            pl.BlockSpec((RB, 16), lambda i: (i, 0)),
        ],
        out_shape=[
            jax.ShapeDtypeStruct((N, 1), jnp.float32),
            jax.ShapeDtypeStruct((N, 128), jnp.float32),
            jax.ShapeDtypeStruct((N, 16), jnp.float32),
        ],
    )(deg2, x, y, w0, lw0)


def _mid_body(relu, s2_ref, gp_ref, dinv_ref, w_ref, b_ref, out_ref):
    dinv = dinv_ref[...]
    u = (s2_ref[0] + s2_ref[1] - gp_ref[...]) * dinv + b_ref[0:1, :]
    if relu:
        u = jnp.maximum(u, 0.0)
    out_ref[...] = jnp.dot(u, w_ref[...],
                           preferred_element_type=jnp.float32) * dinv


def _tc_mid(s2, gprev, dinv, w, b, relu):
    d_in = gprev.shape[1]
    d_out = w.shape[1]
    grid = (N // RB,)
    return pl.pallas_call(
        functools.partial(_mid_body, relu),
        grid=grid,
        in_specs=[
            pl.BlockSpec((2, RB, d_in), lambda i: (0, i, 0)),
            pl.BlockSpec((RB, d_in), lambda i: (i, 0)),
            pl.BlockSpec((RB, 1), lambda i: (i, 0)),
            pl.BlockSpec((d_in, d_out), lambda i: (0, 0)),
            pl.BlockSpec((8, d_in), lambda i: (0, 0)),
        ],
        out_specs=pl.BlockSpec((RB, d_out), lambda i: (i, 0)),
        out_shape=jax.ShapeDtypeStruct((N, d_out), jnp.float32),
    )(s2, gprev, dinv, w, b)


def _fuse_body(sh_ref, ghp_ref, sl_ref, glp_ref, dinv_ref, dw_ref,
               wfh_ref, wfl_ref, wfd_ref, b1_ref, lb9_ref, fb_ref, out_ref):
    dinv = dinv_ref[...]
    h = (sh_ref[0] + sh_ref[1] - ghp_ref[...]) * dinv + b1_ref[0:1, :]
    l = (sl_ref[0] + sl_ref[1] - glp_ref[...]) * dinv + lb9_ref[0:1, :]
    o = (jnp.dot(h, wfh_ref[...], preferred_element_type=jnp.float32)
         + jnp.dot(l, wfl_ref[...], preferred_element_type=jnp.float32)
         + jnp.dot(dw_ref[...], wfd_ref[...], preferred_element_type=jnp.float32)
         + fb_ref[0:1, :])
    out_ref[...] = jax.nn.sigmoid(o)


def _tc_fuse(sh2, ghp, sl2, glp, dinv, dw, wfh, wfl, wfd, b1, lb9, fb):
    grid = (N // RB,)
    return pl.pallas_call(
        _fuse_body,
        grid=grid,
        in_specs=[
            pl.BlockSpec((2, RB, 128), lambda i: (0, i, 0)),
            pl.BlockSpec((RB, 128), lambda i: (i, 0)),
            pl.BlockSpec((2, RB, 16), lambda i: (0, i, 0)),
            pl.BlockSpec((RB, 16), lambda i: (i, 0)),
            pl.BlockSpec((RB, 1), lambda i: (i, 0)),
            pl.BlockSpec((RB, 64), lambda i: (i, 0)),
            pl.BlockSpec((128, 16), lambda i: (0, 0)),
            pl.BlockSpec((16, 16), lambda i: (0, 0)),
            pl.BlockSpec((64, 16), lambda i: (0, 0)),
            pl.BlockSpec((8, 128), lambda i: (0, 0)),
            pl.BlockSpec((8, 16), lambda i: (0, 0)),
            pl.BlockSpec((8, 16), lambda i: (0, 0)),
        ],
        out_specs=pl.BlockSpec((RB, 16), lambda i: (i, 0)),
        out_shape=jax.ShapeDtypeStruct((N, 16), jnp.float32),
    )(sh2, ghp, sl2, glp, dinv, dw, wfh, wfl, wfd, b1, lb9, fb)


def kernel(x, y, edge_index, deep_walk_emb, gcn_W0, gcn_b0, gcn_W1, gcn_b1,
           lbl_W, lbl_b, fusion_W, fusion_b):
    src = edge_index[0].astype(jnp.int32)
    dst = edge_index[1].astype(jnp.int32)
    pad = E_PAD - E
    src_p = jnp.concatenate([src, jnp.zeros((pad,), jnp.int32)])
    dst_p = jnp.concatenate([dst, jnp.full((pad,), N, jnp.int32)])
    src2 = src_p.reshape(E_PAD // 128, 128)
    dst2 = dst_p.reshape(E_PAD // 128, 128)

    deg2 = _deg(dst2)
    dinv, gh, gl = _tc_first(deg2, x, y, gcn_W0, lbl_W[0])

    def tile8(b):
        return jnp.tile(b.reshape(1, -1), (8, 1))

    sh = _agg128(gh, src2, dst2)
    gh2 = _tc_mid(sh, gh, dinv, gcn_W1, tile8(gcn_b0), relu=True)
    sh2 = _agg128(gh2, src2, dst2)

    g = gl
    for j in range(9):
        s = _agg16(g, src2, dst2)
        g = _tc_mid(s, g, dinv, lbl_W[j + 1], tile8(lbl_b[j]), relu=True)
    sl2 = _agg16(g, src2, dst2)

    wfh = fusion_W[:128]
    wfl = fusion_W[128:144]
    wfd = fusion_W[144:]
    return _tc_fuse(sh2, gh2, sl2, g, dinv, deep_walk_emb,
                    wfh, wfl, wfd,
                    tile8(gcn_b1), tile8(lbl_b[9]), tile8(fusion_b))


# speedup vs baseline: 11.3249x; 11.3249x over previous
import functools

import jax
import jax.numpy as jnp
from jax import lax
from jax.experimental import pallas as pl
from jax.experimental.pallas import tpu as pltpu
from jax.experimental.pallas import tpu_sc as plsc

N = 10000
N_PAD = 10240
E = 320000
E_PAD = 327680
NW = 32
EPT = E_PAD // NW
NOUTER = EPT // 1024
RPT = N_PAD // 16
RCH = 128
RB = 2048


def _mesh():
    return plsc.VectorSubcoreMesh(core_axis_name="c", subcore_axis_name="s")


@functools.partial(
    pl.kernel,
    out_type=jax.ShapeDtypeStruct((2, N_PAD, 16), jnp.float32),
    mesh=_mesh(),
    compiler_params=pltpu.CompilerParams(use_tc_tiling_on_sc=False),
    scratch_types=[
        pltpu.VMEM((8, 128), jnp.int32),
        pltpu.VMEM((128, 16), jnp.float32),
        pltpu.VMEM((RCH, 16), jnp.float32),
        pltpu.VMEM_SHARED((N_PAD, 16), jnp.float32),
    ],
)
def _deg(dst_hbm, out_hbm, didx, ones, stage, shared):
    c = lax.axis_index("c")
    s = lax.axis_index("s")
    t = c * 16 + s
    r0 = s * RPT

    def fill_ones(i, _):
        ones[i, :] = jnp.full((16,), 1.0, jnp.float32)
        return 0

    lax.fori_loop(0, 128, fill_ones, 0)

    def fill_zero(i, _):
        stage[i, :] = jnp.zeros((16,), jnp.float32)
        return 0

    lax.fori_loop(0, RCH, fill_zero, 0)

    def zero_blk(k, _):
        pltpu.sync_copy(stage, shared.at[pl.ds(r0 + k * RCH, RCH)])
        return 0

    lax.fori_loop(0, RPT // RCH, zero_blk, 0)
    plsc.subcore_barrier()

    def blk(b, _):
        row0 = t * (EPT // 128) + b * 8
        pltpu.sync_copy(dst_hbm.at[pl.ds(row0, 8)], didx)
        for j in range(8):
            pltpu.sync_copy(ones, shared.at[didx.at[j]], add=True)
        return 0

    lax.fori_loop(0, NOUTER, blk, 0)
    plsc.subcore_barrier()

    def rd(k, _):
        pltpu.sync_copy(shared.at[pl.ds(r0 + k * RCH, RCH)], stage)
        pltpu.sync_copy(stage, out_hbm.at[c, pl.ds(r0 + k * RCH, RCH)])
        return 0

    lax.fori_loop(0, RPT // RCH, rd, 0)


def _make_agg(D):
    @functools.partial(
        pl.kernel,
        out_type=jax.ShapeDtypeStruct((2, N_PAD, D), jnp.float32),
        mesh=_mesh(),
        compiler_params=pltpu.CompilerParams(use_tc_tiling_on_sc=False),
        scratch_types=[
            pltpu.VMEM((8, 128), jnp.int32),
            pltpu.VMEM((8, 128), jnp.int32),
            pltpu.VMEM((128, D), jnp.float32),
            pltpu.VMEM((RCH, D), jnp.float32),
            pltpu.VMEM_SHARED((N_PAD, D), jnp.float32),
            pltpu.SemaphoreType.DMA,
        ],
    )
    def agg(g_hbm, src_hbm, dst_hbm, out_hbm, sidx, didx, rows, stage, shared, sem):
        c = lax.axis_index("c")
        s = lax.axis_index("s")
        t = c * 16 + s
        r0 = s * RPT

        def init_blk(k, _):
            pltpu.sync_copy(g_hbm.at[pl.ds(r0 + k * RCH, RCH)], stage)
            pltpu.sync_copy(stage, shared.at[pl.ds(r0 + k * RCH, RCH)])
            return 0

        lax.fori_loop(0, RPT // RCH, init_blk, 0)
        plsc.subcore_barrier()

        def blk(b, _):
            row0 = t * (EPT // 128) + b * 8
            pltpu.sync_copy(src_hbm.at[pl.ds(row0, 8)], sidx)
            pltpu.sync_copy(dst_hbm.at[pl.ds(row0, 8)], didx)
            for j in range(8):
                pltpu.async_copy(g_hbm.at[sidx.at[j]], rows, sem).wait()
                pltpu.sync_copy(rows, shared.at[didx.at[j]], add=True)
            return 0

        lax.fori_loop(0, NOUTER, blk, 0)
        plsc.subcore_barrier()

        def rd(k, _):
            pltpu.sync_copy(shared.at[pl.ds(r0 + k * RCH, RCH)], stage)
            pltpu.sync_copy(stage, out_hbm.at[c, pl.ds(r0 + k * RCH, RCH)])
            return 0

        lax.fori_loop(0, RPT // RCH, rd, 0)

    return agg


_agg128 = _make_agg(128)
_agg16 = _make_agg(16)


def _first_body(deg_ref, x_ref, y_ref, w0_ref, lw0_ref, dinv_ref, gh_ref, gl_ref):
    d = deg_ref[0, :, 0:1] + deg_ref[1, :, 0:1] + 1.0
    dinv = lax.rsqrt(d)
    dinv_ref[...] = dinv
    gh_ref[...] = jnp.dot(x_ref[...], w0_ref[...],
                          preferred_element_type=jnp.float32) * dinv
    gl_ref[...] = jnp.dot(y_ref[...], lw0_ref[...],
                          preferred_element_type=jnp.float32) * dinv


def _tc_first(deg2, x, y, w0, lw0):
    grid = (N_PAD // RB,)
    return pl.pallas_call(
        _first_body,
        grid=grid,
        in_specs=[
            pl.BlockSpec((2, RB, 16), lambda i: (0, i, 0)),
            pl.BlockSpec((RB, 128), lambda i: (i, 0)),
            pl.BlockSpec((RB, 16), lambda i: (i, 0)),
            pl.BlockSpec((128, 128), lambda i: (0, 0)),
            pl.BlockSpec((16, 16), lambda i: (0, 0)),
        ],
        out_specs=[
            pl.BlockSpec((RB, 1), lambda i: (i, 0)),
            pl.BlockSpec((RB, 128), lambda i: (i, 0)),
            pl.BlockSpec((RB, 16), lambda i: (i, 0)),
        ],
        out_shape=[
            jax.ShapeDtypeStruct((N_PAD, 1), jnp.float32),
            jax.ShapeDtypeStruct((N_PAD, 128), jnp.float32),
            jax.ShapeDtypeStruct((N_PAD, 16), jnp.float32),
        ],
    )(deg2, x, y, w0, lw0)


def _mid_body(relu, s2_ref, gp_ref, dinv_ref, w_ref, b_ref, out_ref):
    dinv = dinv_ref[...]
    u = (s2_ref[0] + s2_ref[1] - gp_ref[...]) * dinv + b_ref[0:1, :]
    if relu:
        u = jnp.maximum(u, 0.0)
    out_ref[...] = jnp.dot(u, w_ref[...],
                           preferred_element_type=jnp.float32) * dinv


def _tc_mid(s2, gprev, dinv, w, b, relu):
    d_in = gprev.shape[1]
    d_out = w.shape[1]
    grid = (N_PAD // RB,)
    return pl.pallas_call(
        functools.partial(_mid_body, relu),
        grid=grid,
        in_specs=[
            pl.BlockSpec((2, RB, d_in), lambda i: (0, i, 0)),
            pl.BlockSpec((RB, d_in), lambda i: (i, 0)),
            pl.BlockSpec((RB, 1), lambda i: (i, 0)),
            pl.BlockSpec((d_in, d_out), lambda i: (0, 0)),
            pl.BlockSpec((8, d_in), lambda i: (0, 0)),
        ],
        out_specs=pl.BlockSpec((RB, d_out), lambda i: (i, 0)),
        out_shape=jax.ShapeDtypeStruct((N_PAD, d_out), jnp.float32),
    )(s2, gprev, dinv, w, b)


def _fuse_body(sh_ref, ghp_ref, sl_ref, glp_ref, dinv_ref, dw_ref,
               wfh_ref, wfl_ref, wfd_ref, b1_ref, lb9_ref, fb_ref, out_ref):
    dinv = dinv_ref[...]
    h = (sh_ref[0] + sh_ref[1] - ghp_ref[...]) * dinv + b1_ref[0:1, :]
    l = (sl_ref[0] + sl_ref[1] - glp_ref[...]) * dinv + lb9_ref[0:1, :]
    o = (jnp.dot(h, wfh_ref[...], preferred_element_type=jnp.float32)
         + jnp.dot(l, wfl_ref[...], preferred_element_type=jnp.float32)
         + jnp.dot(dw_ref[...], wfd_ref[...], preferred_element_type=jnp.float32)
         + fb_ref[0:1, :])
    out_ref[...] = jax.nn.sigmoid(o)


def _tc_fuse(sh2, ghp, sl2, glp, dinv, dw, wfh, wfl, wfd, b1, lb9, fb):
    grid = (N_PAD // RB,)
    return pl.pallas_call(
        _fuse_body,
        grid=grid,
        in_specs=[
            pl.BlockSpec((2, RB, 128), lambda i: (0, i, 0)),
            pl.BlockSpec((RB, 128), lambda i: (i, 0)),
            pl.BlockSpec((2, RB, 16), lambda i: (0, i, 0)),
            pl.BlockSpec((RB, 16), lambda i: (i, 0)),
            pl.BlockSpec((RB, 1), lambda i: (i, 0)),
            pl.BlockSpec((RB, 64), lambda i: (i, 0)),
            pl.BlockSpec((128, 16), lambda i: (0, 0)),
            pl.BlockSpec((16, 16), lambda i: (0, 0)),
            pl.BlockSpec((64, 16), lambda i: (0, 0)),
            pl.BlockSpec((8, 128), lambda i: (0, 0)),
            pl.BlockSpec((8, 16), lambda i: (0, 0)),
            pl.BlockSpec((8, 16), lambda i: (0, 0)),
        ],
        out_specs=pl.BlockSpec((RB, 16), lambda i: (i, 0)),
        out_shape=jax.ShapeDtypeStruct((N_PAD, 16), jnp.float32),
    )(sh2, ghp, sl2, glp, dinv, dw, wfh, wfl, wfd, b1, lb9, fb)


def kernel(x, y, edge_index, deep_walk_emb, gcn_W0, gcn_b0, gcn_W1, gcn_b1,
           lbl_W, lbl_b, fusion_W, fusion_b):
    src = edge_index[0].astype(jnp.int32)
    dst = edge_index[1].astype(jnp.int32)
    pad = E_PAD - E
    src_p = jnp.concatenate([src, jnp.zeros((pad,), jnp.int32)])
    dst_p = jnp.concatenate([dst, jnp.full((pad,), N, jnp.int32)])
    src2 = src_p.reshape(E_PAD // 128, 128)
    dst2 = dst_p.reshape(E_PAD // 128, 128)

    rpad = N_PAD - N
    x_p = jnp.concatenate([x, jnp.zeros((rpad, x.shape[1]), x.dtype)])
    y_p = jnp.concatenate([y, jnp.zeros((rpad, y.shape[1]), y.dtype)])
    dw_p = jnp.concatenate(
        [deep_walk_emb, jnp.zeros((rpad, deep_walk_emb.shape[1]), deep_walk_emb.dtype)])

    deg2 = _deg(dst2)
    dinv, gh, gl = _tc_first(deg2, x_p, y_p, gcn_W0, lbl_W[0])

    def tile8(b):
        return jnp.tile(b.reshape(1, -1), (8, 1))

    sh = _agg128(gh, src2, dst2)
    gh2 = _tc_mid(sh, gh, dinv, gcn_W1, tile8(gcn_b0), relu=True)
    sh2 = _agg128(gh2, src2, dst2)

    g = gl
    for j in range(9):
        s = _agg16(g, src2, dst2)
        g = _tc_mid(s, g, dinv, lbl_W[j + 1], tile8(lbl_b[j]), relu=True)
    sl2 = _agg16(g, src2, dst2)

    wfh = fusion_W[:128]
    wfl = fusion_W[128:144]
    wfd = fusion_W[144:]
    out = _tc_fuse(sh2, gh2, sl2, g, dinv, dw_p,
                   wfh, wfl, wfd,
                   tile8(gcn_b1), tile8(lbl_b[9]), tile8(fusion_b))
    return out[:N]


# speedup vs baseline: 12.4542x; 1.0997x over previous
import functools

import jax
import jax.numpy as jnp
from jax import lax
from jax.experimental import pallas as pl
from jax.experimental.pallas import tpu as pltpu
from jax.experimental.pallas import tpu_sc as plsc

N = 10000
N_PAD = 10240
E = 320000
E_PAD = 327680
NW = 32
EPT = E_PAD // NW
NOUTER = EPT // 1024
RPT = N_PAD // 16
RCH = 128
RB = 2048


def _mesh():
    return plsc.VectorSubcoreMesh(core_axis_name="c", subcore_axis_name="s")


@functools.partial(
    pl.kernel,
    out_type=jax.ShapeDtypeStruct((2, N_PAD, 16), jnp.float32),
    mesh=_mesh(),
    compiler_params=pltpu.CompilerParams(use_tc_tiling_on_sc=False),
    scratch_types=[
        pltpu.VMEM((EPT // 128, 128), jnp.int32),
        pltpu.VMEM((128, 16), jnp.float32),
        pltpu.VMEM((RCH, 16), jnp.float32),
        pltpu.VMEM_SHARED((N_PAD, 16), jnp.float32),
        pltpu.SemaphoreType.DMA,
    ],
)
def _deg(dst_hbm, out_hbm, didx, ones, stage, shared, sem):
    c = lax.axis_index("c")
    s = lax.axis_index("s")
    t = c * 16 + s
    r0 = s * RPT

    def fill_ones(i, _):
        ones[i, :] = jnp.full((16,), 1.0, jnp.float32)
        return 0

    lax.fori_loop(0, 128, fill_ones, 0)

    def fill_zero(i, _):
        stage[i, :] = jnp.zeros((16,), jnp.float32)
        return 0

    lax.fori_loop(0, RCH, fill_zero, 0)

    def zero_blk(k, _):
        pltpu.sync_copy(stage, shared.at[pl.ds(r0 + k * RCH, RCH)])
        return 0

    lax.fori_loop(0, RPT // RCH, zero_blk, 0)
    plsc.subcore_barrier()

    pltpu.sync_copy(dst_hbm.at[pl.ds(t * (EPT // 128), EPT // 128)], didx)

    def blk(b, _):
        descs = [pltpu.async_copy(ones, shared.at[didx.at[b * 8 + j]], sem, add=True)
                 for j in range(8)]
        for d in descs:
            d.wait()
        return 0

    lax.fori_loop(0, NOUTER, blk, 0)
    plsc.subcore_barrier()

    def rd(k, _):
        pltpu.sync_copy(shared.at[pl.ds(r0 + k * RCH, RCH)], stage)
        pltpu.sync_copy(stage, out_hbm.at[c, pl.ds(r0 + k * RCH, RCH)])
        return 0

    lax.fori_loop(0, RPT // RCH, rd, 0)


def _make_agg(D, BLK):
    nb = EPT // BLK
    nch = RPT // BLK

    @functools.partial(
        pl.kernel,
        out_type=jax.ShapeDtypeStruct((2, N_PAD, D), jnp.float32),
        mesh=_mesh(),
        compiler_params=pltpu.CompilerParams(use_tc_tiling_on_sc=False),
        scratch_types=[
            pltpu.VMEM((nb, BLK), jnp.int32),
            pltpu.VMEM((nb, BLK), jnp.int32),
            pltpu.VMEM((BLK, D), jnp.float32),
            pltpu.VMEM((BLK, D), jnp.float32),
            pltpu.VMEM_SHARED((N_PAD, D), jnp.float32),
            pltpu.SemaphoreType.DMA,
            pltpu.SemaphoreType.DMA,
        ],
    )
    def agg(g_hbm, src_hbm, dst_hbm, out_hbm, sidx, didx, rows0, rows1,
            shared, sem0, sem1):
        c = lax.axis_index("c")
        s = lax.axis_index("s")
        t = c * 16 + s
        r0 = s * RPT

        def init_blk(k, _):
            pltpu.sync_copy(g_hbm.at[pl.ds(r0 + k * BLK, BLK)], rows0)
            pltpu.sync_copy(rows0, shared.at[pl.ds(r0 + k * BLK, BLK)])
            return 0

        lax.fori_loop(0, nch, init_blk, 0)
        pltpu.sync_copy(src_hbm.at[pl.ds(t * nb, nb)], sidx)
        pltpu.sync_copy(dst_hbm.at[pl.ds(t * nb, nb)], didx)
        plsc.subcore_barrier()

        pltpu.async_copy(g_hbm.at[sidx.at[0]], rows0, sem0)

        def blk(p, _):
            pltpu.make_async_copy(g_hbm.at[sidx.at[2 * p]], rows0, sem0).wait()
            pltpu.async_copy(g_hbm.at[sidx.at[2 * p + 1]], rows1, sem1)
            pltpu.sync_copy(rows0, shared.at[didx.at[2 * p]], add=True)
            pltpu.make_async_copy(g_hbm.at[sidx.at[2 * p + 1]], rows1, sem1).wait()

            @pl.when(p < nb // 2 - 1)
            def _():
                pltpu.async_copy(g_hbm.at[sidx.at[2 * p + 2]], rows0, sem0)

            pltpu.sync_copy(rows1, shared.at[didx.at[2 * p + 1]], add=True)
            return 0

        lax.fori_loop(0, nb // 2, blk, 0)
        plsc.subcore_barrier()

        def rd(k, _):
            pltpu.sync_copy(shared.at[pl.ds(r0 + k * BLK, BLK)], rows0)
            pltpu.sync_copy(rows0, out_hbm.at[c, pl.ds(r0 + k * BLK, BLK)])
            return 0

        lax.fori_loop(0, nch, rd, 0)

    return agg


_agg128 = _make_agg(128, 64)
_agg16 = _make_agg(16, 128)


def _first_body(deg_ref, x_ref, y_ref, w0_ref, lw0_ref, dinv_ref, gh_ref, gl_ref):
    d = deg_ref[0, :, 0:1] + deg_ref[1, :, 0:1] + 1.0
    dinv = lax.rsqrt(d)
    dinv_ref[...] = dinv
    gh_ref[...] = jnp.dot(x_ref[...], w0_ref[...],
                          preferred_element_type=jnp.float32) * dinv
    gl_ref[...] = jnp.dot(y_ref[...], lw0_ref[...],
                          preferred_element_type=jnp.float32) * dinv


def _tc_first(deg2, x, y, w0, lw0):
    grid = (N_PAD // RB,)
    return pl.pallas_call(
        _first_body,
        grid=grid,
        in_specs=[
            pl.BlockSpec((2, RB, 16), lambda i: (0, i, 0)),
            pl.BlockSpec((RB, 128), lambda i: (i, 0)),
            pl.BlockSpec((RB, 16), lambda i: (i, 0)),
            pl.BlockSpec((128, 128), lambda i: (0, 0)),
            pl.BlockSpec((16, 16), lambda i: (0, 0)),
        ],
        out_specs=[
            pl.BlockSpec((RB, 1), lambda i: (i, 0)),
            pl.BlockSpec((RB, 128), lambda i: (i, 0)),
            pl.BlockSpec((RB, 16), lambda i: (i, 0)),
        ],
        out_shape=[
            jax.ShapeDtypeStruct((N_PAD, 1), jnp.float32),
            jax.ShapeDtypeStruct((N_PAD, 128), jnp.float32),
            jax.ShapeDtypeStruct((N_PAD, 16), jnp.float32),
        ],
    )(deg2, x, y, w0, lw0)


def _mid_body(relu, s2_ref, gp_ref, dinv_ref, w_ref, b_ref, out_ref):
    dinv = dinv_ref[...]
    u = (s2_ref[0] + s2_ref[1] - gp_ref[...]) * dinv + b_ref[0:1, :]
    if relu:
        u = jnp.maximum(u, 0.0)
    out_ref[...] = jnp.dot(u, w_ref[...],
                           preferred_element_type=jnp.float32) * dinv


def _tc_mid(s2, gprev, dinv, w, b, relu):
    d_in = gprev.shape[1]
    d_out = w.shape[1]
    grid = (N_PAD // RB,)
    return pl.pallas_call(
        functools.partial(_mid_body, relu),
        grid=grid,
        in_specs=[
            pl.BlockSpec((2, RB, d_in), lambda i: (0, i, 0)),
            pl.BlockSpec((RB, d_in), lambda i: (i, 0)),
            pl.BlockSpec((RB, 1), lambda i: (i, 0)),
            pl.BlockSpec((d_in, d_out), lambda i: (0, 0)),
            pl.BlockSpec((8, d_in), lambda i: (0, 0)),
        ],
        out_specs=pl.BlockSpec((RB, d_out), lambda i: (i, 0)),
        out_shape=jax.ShapeDtypeStruct((N_PAD, d_out), jnp.float32),
    )(s2, gprev, dinv, w, b)


def _fuse_body(sh_ref, ghp_ref, sl_ref, glp_ref, dinv_ref, dw_ref,
               wfh_ref, wfl_ref, wfd_ref, b1_ref, lb9_ref, fb_ref, out_ref):
    dinv = dinv_ref[...]
    h = (sh_ref[0] + sh_ref[1] - ghp_ref[...]) * dinv + b1_ref[0:1, :]
    l = (sl_ref[0] + sl_ref[1] - glp_ref[...]) * dinv + lb9_ref[0:1, :]
    o = (jnp.dot(h, wfh_ref[...], preferred_element_type=jnp.float32)
         + jnp.dot(l, wfl_ref[...], preferred_element_type=jnp.float32)
         + jnp.dot(dw_ref[...], wfd_ref[...], preferred_element_type=jnp.float32)
         + fb_ref[0:1, :])
    out_ref[...] = jax.nn.sigmoid(o)


def _tc_fuse(sh2, ghp, sl2, glp, dinv, dw, wfh, wfl, wfd, b1, lb9, fb):
    grid = (N_PAD // RB,)
    return pl.pallas_call(
        _fuse_body,
        grid=grid,
        in_specs=[
            pl.BlockSpec((2, RB, 128), lambda i: (0, i, 0)),
            pl.BlockSpec((RB, 128), lambda i: (i, 0)),
            pl.BlockSpec((2, RB, 16), lambda i: (0, i, 0)),
            pl.BlockSpec((RB, 16), lambda i: (i, 0)),
            pl.BlockSpec((RB, 1), lambda i: (i, 0)),
            pl.BlockSpec((RB, 64), lambda i: (i, 0)),
            pl.BlockSpec((128, 16), lambda i: (0, 0)),
            pl.BlockSpec((16, 16), lambda i: (0, 0)),
            pl.BlockSpec((64, 16), lambda i: (0, 0)),
            pl.BlockSpec((8, 128), lambda i: (0, 0)),
            pl.BlockSpec((8, 16), lambda i: (0, 0)),
            pl.BlockSpec((8, 16), lambda i: (0, 0)),
        ],
        out_specs=pl.BlockSpec((RB, 16), lambda i: (i, 0)),
        out_shape=jax.ShapeDtypeStruct((N_PAD, 16), jnp.float32),
    )(sh2, ghp, sl2, glp, dinv, dw, wfh, wfl, wfd, b1, lb9, fb)


def kernel(x, y, edge_index, deep_walk_emb, gcn_W0, gcn_b0, gcn_W1, gcn_b1,
           lbl_W, lbl_b, fusion_W, fusion_b):
    src = edge_index[0].astype(jnp.int32)
    dst = edge_index[1].astype(jnp.int32)
    pad = E_PAD - E
    src_p = jnp.concatenate([src, jnp.zeros((pad,), jnp.int32)])
    dst_p = jnp.concatenate([dst, jnp.full((pad,), N, jnp.int32)])
    src2 = src_p.reshape(E_PAD // 128, 128)
    dst2 = dst_p.reshape(E_PAD // 128, 128)
    src2h = src_p.reshape(E_PAD // 64, 64)
    dst2h = dst_p.reshape(E_PAD // 64, 64)

    rpad = N_PAD - N
    x_p = jnp.concatenate([x, jnp.zeros((rpad, x.shape[1]), x.dtype)])
    y_p = jnp.concatenate([y, jnp.zeros((rpad, y.shape[1]), y.dtype)])
    dw_p = jnp.concatenate(
        [deep_walk_emb, jnp.zeros((rpad, deep_walk_emb.shape[1]), deep_walk_emb.dtype)])

    deg2 = _deg(dst2)
    dinv, gh, gl = _tc_first(deg2, x_p, y_p, gcn_W0, lbl_W[0])

    def tile8(b):
        return jnp.tile(b.reshape(1, -1), (8, 1))

    sh = _agg128(gh, src2h, dst2h)
    gh2 = _tc_mid(sh, gh, dinv, gcn_W1, tile8(gcn_b0), relu=True)
    sh2 = _agg128(gh2, src2h, dst2h)

    g = gl
    for j in range(9):
        s = _agg16(g, src2, dst2)
        g = _tc_mid(s, g, dinv, lbl_W[j + 1], tile8(lbl_b[j]), relu=True)
    sl2 = _agg16(g, src2, dst2)

    wfh = fusion_W[:128]
    wfl = fusion_W[128:144]
    wfd = fusion_W[144:]
    out = _tc_fuse(sh2, gh2, sl2, g, dinv, dw_p,
                   wfh, wfl, wfd,
                   tile8(gcn_b1), tile8(lbl_b[9]), tile8(fusion_b))
    return out[:N]


# speedup vs baseline: 19.9539x; 1.6022x over previous
import functools

import jax
import jax.numpy as jnp
from jax import lax
from jax.experimental import pallas as pl
from jax.experimental.pallas import tpu as pltpu
from jax.experimental.pallas import tpu_sc as plsc

N = 10000
N_PAD = 10240
E = 320000
E_PAD = 327680
NW = 32
EPT = E_PAD // NW
NOUTER = EPT // 1024
RPT = N_PAD // 16
RCH = 128
RB = 2048


def _mesh():
    return plsc.VectorSubcoreMesh(core_axis_name="c", subcore_axis_name="s")


@functools.partial(
    pl.kernel,
    out_type=jax.ShapeDtypeStruct((2, N_PAD, 16), jnp.float32),
    mesh=_mesh(),
    compiler_params=pltpu.CompilerParams(use_tc_tiling_on_sc=False),
    scratch_types=[
        pltpu.VMEM((EPT // 128, 128), jnp.int32),
        pltpu.VMEM((128, 16), jnp.float32),
        pltpu.VMEM((RCH, 16), jnp.float32),
        pltpu.VMEM_SHARED((N_PAD, 16), jnp.float32),
        pltpu.SemaphoreType.DMA,
    ],
)
def _deg(dst_hbm, out_hbm, didx, ones, stage, shared, sem):
    c = lax.axis_index("c")
    s = lax.axis_index("s")
    t = c * 16 + s
    r0 = s * RPT

    def fill_ones(i, _):
        ones[i, :] = jnp.full((16,), 1.0, jnp.float32)
        return 0

    lax.fori_loop(0, 128, fill_ones, 0)

    def fill_zero(i, _):
        stage[i, :] = jnp.zeros((16,), jnp.float32)
        return 0

    lax.fori_loop(0, RCH, fill_zero, 0)

    def zero_blk(k, _):
        pltpu.sync_copy(stage, shared.at[pl.ds(r0 + k * RCH, RCH)])
        return 0

    lax.fori_loop(0, RPT // RCH, zero_blk, 0)
    plsc.subcore_barrier()

    pltpu.sync_copy(dst_hbm.at[pl.ds(t * (EPT // 128), EPT // 128)], didx)

    def blk(b, _):
        descs = [pltpu.async_copy(ones, shared.at[didx.at[b * 8 + j]], sem, add=True)
                 for j in range(8)]
        for d in descs:
            d.wait()
        return 0

    lax.fori_loop(0, NOUTER, blk, 0)
    plsc.subcore_barrier()

    def rd(k, _):
        pltpu.sync_copy(shared.at[pl.ds(r0 + k * RCH, RCH)], stage)
        pltpu.sync_copy(stage, out_hbm.at[c, pl.ds(r0 + k * RCH, RCH)])
        return 0

    lax.fori_loop(0, RPT // RCH, rd, 0)


def _make_agg(D, BLK):
    nb = EPT // BLK
    nch = RPT // BLK

    @functools.partial(
        pl.kernel,
        out_type=jax.ShapeDtypeStruct((2, N_PAD, D), jnp.float32),
        mesh=_mesh(),
        compiler_params=pltpu.CompilerParams(use_tc_tiling_on_sc=False),
        scratch_types=[
            pltpu.VMEM((nb, BLK), jnp.int32),
            pltpu.VMEM((nb, BLK), jnp.int32),
            pltpu.VMEM((BLK, D), jnp.float32),
            pltpu.VMEM((BLK, D), jnp.float32),
            pltpu.VMEM_SHARED((N_PAD, D), jnp.float32),
            pltpu.SemaphoreType.DMA,
            pltpu.SemaphoreType.DMA,
        ],
    )
    def agg(g_hbm, src_hbm, dst_hbm, out_hbm, sidx, didx, rows0, rows1,
            shared, sem0, sem1):
        c = lax.axis_index("c")
        s = lax.axis_index("s")
        t = c * 16 + s
        r0 = s * RPT

        def init_blk(k, _):
            pltpu.sync_copy(g_hbm.at[pl.ds(r0 + k * BLK, BLK)], rows0)
            pltpu.sync_copy(rows0, shared.at[pl.ds(r0 + k * BLK, BLK)])
            return 0

        lax.fori_loop(0, nch, init_blk, 0)
        pltpu.sync_copy(src_hbm.at[pl.ds(t * nb, nb)], sidx)
        pltpu.sync_copy(dst_hbm.at[pl.ds(t * nb, nb)], didx)
        plsc.subcore_barrier()

        pltpu.async_copy(g_hbm.at[sidx.at[0]], rows0, sem0)

        def blk(p, _):
            pltpu.make_async_copy(g_hbm.at[sidx.at[2 * p]], rows0, sem0).wait()
            pltpu.async_copy(g_hbm.at[sidx.at[2 * p + 1]], rows1, sem1)
            pltpu.sync_copy(rows0, shared.at[didx.at[2 * p]], add=True)
            pltpu.make_async_copy(g_hbm.at[sidx.at[2 * p + 1]], rows1, sem1).wait()

            @pl.when(p < nb // 2 - 1)
            def _():
                pltpu.async_copy(g_hbm.at[sidx.at[2 * p + 2]], rows0, sem0)

            pltpu.sync_copy(rows1, shared.at[didx.at[2 * p + 1]], add=True)
            return 0

        lax.fori_loop(0, nb // 2, blk, 0)
        plsc.subcore_barrier()

        def rd(k, _):
            pltpu.sync_copy(shared.at[pl.ds(r0 + k * BLK, BLK)], rows0)
            pltpu.sync_copy(rows0, out_hbm.at[c, pl.ds(r0 + k * BLK, BLK)])
            return 0

        lax.fori_loop(0, nch, rd, 0)

    return agg


_agg128 = _make_agg(128, 64)
_agg16 = _make_agg(16, 128)

HBLK = 64
HEPT = E_PAD // 16


@functools.partial(
    pl.kernel,
    out_type=jax.ShapeDtypeStruct((N_PAD, 128), jnp.float32),
    mesh=_mesh(),
    compiler_params=pltpu.CompilerParams(use_tc_tiling_on_sc=False),
    scratch_types=[
        pltpu.VMEM((HEPT // 2 // HBLK, HBLK), jnp.int32),
        pltpu.VMEM((HEPT // 2 // HBLK, HBLK), jnp.int32),
        pltpu.VMEM((HBLK, 64), jnp.float32),
        pltpu.VMEM((HBLK, 64), jnp.float32),
        pltpu.VMEM_SHARED((N_PAD, 64), jnp.float32),
        pltpu.VMEM_SHARED((N_PAD, 64), jnp.float32),
        pltpu.SemaphoreType.DMA,
        pltpu.SemaphoreType.DMA,
    ],
)
def _agg128f(g_hbm, src_hbm, dst_hbm, out_hbm, sidx, didx, rows0, rows1,
             gtab, acc, sem0, sem1):
    c = lax.axis_index("c")
    s = lax.axis_index("s")
    r0 = s * RPT
    nbh = HEPT // 2 // HBLK

    def init_blk(k, _):
        rr = r0 + k * HBLK
        pltpu.sync_copy(g_hbm.at[pl.ds(rr, HBLK), pl.ds(c * 64, 64)], rows0)
        pltpu.sync_copy(rows0, gtab.at[pl.ds(rr, HBLK)])
        pltpu.sync_copy(rows0, acc.at[pl.ds(rr, HBLK)])
        return 0

    lax.fori_loop(0, RPT // HBLK, init_blk, 0)

    def half(h):
        base = s * (HEPT // HBLK) + h * nbh
        pltpu.sync_copy(src_hbm.at[pl.ds(base, nbh)], sidx)
        pltpu.sync_copy(dst_hbm.at[pl.ds(base, nbh)], didx)
        plsc.subcore_barrier()
        pltpu.async_copy(gtab.at[sidx.at[0]], rows0, sem0)

        def blk(p, _):
            pltpu.make_async_copy(gtab.at[sidx.at[2 * p]], rows0, sem0).wait()
            pltpu.async_copy(gtab.at[sidx.at[2 * p + 1]], rows1, sem1)
            pltpu.sync_copy(rows0, acc.at[didx.at[2 * p]], add=True)
            pltpu.make_async_copy(gtab.at[sidx.at[2 * p + 1]], rows1, sem1).wait()

            @pl.when(p < nbh // 2 - 1)
            def _():
                pltpu.async_copy(gtab.at[sidx.at[2 * p + 2]], rows0, sem0)

            pltpu.sync_copy(rows1, acc.at[didx.at[2 * p + 1]], add=True)
            return 0

        lax.fori_loop(0, nbh // 2, blk, 0)

    half(0)
    half(1)
    plsc.subcore_barrier()

    def rd(k, _):
        rr = r0 + k * HBLK
        pltpu.sync_copy(acc.at[pl.ds(rr, HBLK)], rows0)
        pltpu.sync_copy(rows0, out_hbm.at[pl.ds(rr, HBLK), pl.ds(c * 64, 64)])
        return 0

    lax.fori_loop(0, RPT // HBLK, rd, 0)


def _first_body(deg_ref, x_ref, y_ref, w0_ref, lw0_ref, dinv_ref, gh_ref, gl_ref):
    d = deg_ref[0, :, 0:1] + deg_ref[1, :, 0:1] + 1.0
    dinv = lax.rsqrt(d)
    dinv_ref[...] = dinv
    gh_ref[...] = jnp.dot(x_ref[...], w0_ref[...],
                          preferred_element_type=jnp.float32) * dinv
    gl_ref[...] = jnp.dot(y_ref[...], lw0_ref[...],
                          preferred_element_type=jnp.float32) * dinv


def _tc_first(deg2, x, y, w0, lw0):
    grid = (N_PAD // RB,)
    return pl.pallas_call(
        _first_body,
        grid=grid,
        in_specs=[
            pl.BlockSpec((2, RB, 16), lambda i: (0, i, 0)),
            pl.BlockSpec((RB, 128), lambda i: (i, 0)),
            pl.BlockSpec((RB, 16), lambda i: (i, 0)),
            pl.BlockSpec((128, 128), lambda i: (0, 0)),
            pl.BlockSpec((16, 16), lambda i: (0, 0)),
        ],
        out_specs=[
            pl.BlockSpec((RB, 1), lambda i: (i, 0)),
            pl.BlockSpec((RB, 128), lambda i: (i, 0)),
            pl.BlockSpec((RB, 16), lambda i: (i, 0)),
        ],
        out_shape=[
            jax.ShapeDtypeStruct((N_PAD, 1), jnp.float32),
            jax.ShapeDtypeStruct((N_PAD, 128), jnp.float32),
            jax.ShapeDtypeStruct((N_PAD, 16), jnp.float32),
        ],
    )(deg2, x, y, w0, lw0)


def _mid_body(relu, s2_ref, gp_ref, dinv_ref, w_ref, b_ref, out_ref):
    dinv = dinv_ref[...]
    u = (s2_ref[0] + s2_ref[1] - gp_ref[...]) * dinv + b_ref[0:1, :]
    if relu:
        u = jnp.maximum(u, 0.0)
    out_ref[...] = jnp.dot(u, w_ref[...],
                           preferred_element_type=jnp.float32) * dinv


def _tc_mid(s2, gprev, dinv, w, b, relu):
    d_in = gprev.shape[1]
    d_out = w.shape[1]
    grid = (N_PAD // RB,)
    return pl.pallas_call(
        functools.partial(_mid_body, relu),
        grid=grid,
        in_specs=[
            pl.BlockSpec((2, RB, d_in), lambda i: (0, i, 0)),
            pl.BlockSpec((RB, d_in), lambda i: (i, 0)),
            pl.BlockSpec((RB, 1), lambda i: (i, 0)),
            pl.BlockSpec((d_in, d_out), lambda i: (0, 0)),
            pl.BlockSpec((8, d_in), lambda i: (0, 0)),
        ],
        out_specs=pl.BlockSpec((RB, d_out), lambda i: (i, 0)),
        out_shape=jax.ShapeDtypeStruct((N_PAD, d_out), jnp.float32),
    )(s2, gprev, dinv, w, b)


def _mid2_body(relu, s_ref, dinv_ref, w_ref, b_ref, out_ref):
    dinv = dinv_ref[...]
    u = s_ref[...] * dinv + b_ref[0:1, :]
    if relu:
        u = jnp.maximum(u, 0.0)
    out_ref[...] = jnp.dot(u, w_ref[...],
                           preferred_element_type=jnp.float32) * dinv


def _tc_mid2(s, dinv, w, b, relu):
    d_in = s.shape[1]
    d_out = w.shape[1]
    grid = (N_PAD // RB,)
    return pl.pallas_call(
        functools.partial(_mid2_body, relu),
        grid=grid,
        in_specs=[
            pl.BlockSpec((RB, d_in), lambda i: (i, 0)),
            pl.BlockSpec((RB, 1), lambda i: (i, 0)),
            pl.BlockSpec((d_in, d_out), lambda i: (0, 0)),
            pl.BlockSpec((8, d_in), lambda i: (0, 0)),
        ],
        out_specs=pl.BlockSpec((RB, d_out), lambda i: (i, 0)),
        out_shape=jax.ShapeDtypeStruct((N_PAD, d_out), jnp.float32),
    )(s, dinv, w, b)


def _fuse_body(sh_ref, sl_ref, glp_ref, dinv_ref, dw_ref,
               wfh_ref, wfl_ref, wfd_ref, b1_ref, lb9_ref, fb_ref, out_ref):
    dinv = dinv_ref[...]
    h = sh_ref[...] * dinv + b1_ref[0:1, :]
    l = (sl_ref[0] + sl_ref[1] - glp_ref[...]) * dinv + lb9_ref[0:1, :]
    o = (jnp.dot(h, wfh_ref[...], preferred_element_type=jnp.float32)
         + jnp.dot(l, wfl_ref[...], preferred_element_type=jnp.float32)
         + jnp.dot(dw_ref[...], wfd_ref[...], preferred_element_type=jnp.float32)
         + fb_ref[0:1, :])
    out_ref[...] = jax.nn.sigmoid(o)


def _tc_fuse(sh2, sl2, glp, dinv, dw, wfh, wfl, wfd, b1, lb9, fb):
    grid = (N_PAD // RB,)
    return pl.pallas_call(
        _fuse_body,
        grid=grid,
        in_specs=[
            pl.BlockSpec((RB, 128), lambda i: (i, 0)),
            pl.BlockSpec((2, RB, 16), lambda i: (0, i, 0)),
            pl.BlockSpec((RB, 16), lambda i: (i, 0)),
            pl.BlockSpec((RB, 1), lambda i: (i, 0)),
            pl.BlockSpec((RB, 64), lambda i: (i, 0)),
            pl.BlockSpec((128, 16), lambda i: (0, 0)),
            pl.BlockSpec((16, 16), lambda i: (0, 0)),
            pl.BlockSpec((64, 16), lambda i: (0, 0)),
            pl.BlockSpec((8, 128), lambda i: (0, 0)),
            pl.BlockSpec((8, 16), lambda i: (0, 0)),
            pl.BlockSpec((8, 16), lambda i: (0, 0)),
        ],
        out_specs=pl.BlockSpec((RB, 16), lambda i: (i, 0)),
        out_shape=jax.ShapeDtypeStruct((N_PAD, 16), jnp.float32),
    )(sh2, sl2, glp, dinv, dw, wfh, wfl, wfd, b1, lb9, fb)


def kernel(x, y, edge_index, deep_walk_emb, gcn_W0, gcn_b0, gcn_W1, gcn_b1,
           lbl_W, lbl_b, fusion_W, fusion_b):
    src = edge_index[0].astype(jnp.int32)
    dst = edge_index[1].astype(jnp.int32)
    pad = E_PAD - E
    src_p = jnp.concatenate([src, jnp.zeros((pad,), jnp.int32)])
    dst_p = jnp.concatenate([dst, jnp.full((pad,), N, jnp.int32)])
    src2 = src_p.reshape(E_PAD // 128, 128)
    dst2 = dst_p.reshape(E_PAD // 128, 128)
    src2h = src_p.reshape(E_PAD // 64, 64)
    dst2h = dst_p.reshape(E_PAD // 64, 64)

    rpad = N_PAD - N
    x_p = jnp.concatenate([x, jnp.zeros((rpad, x.shape[1]), x.dtype)])
    y_p = jnp.concatenate([y, jnp.zeros((rpad, y.shape[1]), y.dtype)])
    dw_p = jnp.concatenate(
        [deep_walk_emb, jnp.zeros((rpad, deep_walk_emb.shape[1]), deep_walk_emb.dtype)])

    deg2 = _deg(dst2)
    dinv, gh, gl = _tc_first(deg2, x_p, y_p, gcn_W0, lbl_W[0])

    def tile8(b):
        return jnp.tile(b.reshape(1, -1), (8, 1))

    sh = _agg128f(gh, src2h, dst2h)
    gh2 = _tc_mid2(sh, dinv, gcn_W1, tile8(gcn_b0), relu=True)
    sh2 = _agg128f(gh2, src2h, dst2h)

    g = gl
    for j in range(9):
        s = _agg16(g, src2, dst2)
        g = _tc_mid(s, g, dinv, lbl_W[j + 1], tile8(lbl_b[j]), relu=True)
    sl2 = _agg16(g, src2, dst2)

    wfh = fusion_W[:128]
    wfl = fusion_W[128:144]
    wfd = fusion_W[144:]
    out = _tc_fuse(sh2, sl2, g, dinv, dw_p,
                   wfh, wfl, wfd,
                   tile8(gcn_b1), tile8(lbl_b[9]), tile8(fusion_b))
    return out[:N]
